# baseline (speedup 1.0000x reference)
import functools

import jax
import jax.numpy as jnp
from jax import lax
from jax.experimental import pallas as pl
from jax.experimental.pallas import tpu as pltpu
from jax.experimental.pallas import tpu_sc as plsc

MAX_LEN = 4096
D_MODEL = 768
BATCH = 4
NCOND = 3

NC, NS, NLANES = 2, 16, 16
NW = NC * NS
ROWS_PER_W = MAX_LEN // NW
CHUNK = 32
NCHUNK = ROWS_PER_W // CHUNK

_sc_mesh = plsc.VectorSubcoreMesh(core_axis_name="c", subcore_axis_name="s")


@functools.partial(
    pl.kernel,
    out_type=jax.ShapeDtypeStruct((MAX_LEN, D_MODEL), jnp.float32),
    mesh=_sc_mesh,
    scratch_types=[
        pltpu.VMEM((ROWS_PER_W + NLANES,), jnp.int32),
        pltpu.VMEM((NCOND, D_MODEL), jnp.float32),
        pltpu.VMEM((CHUNK, D_MODEL), jnp.float32),
    ],
)
def _sc_combine(idx_hbm, pos_hbm, ctab_hbm, comb_hbm, idx_v, ctab_v, pos_v):
    wid = lax.axis_index("s") * NC + lax.axis_index("c")
    base = wid * ROWS_PER_W
    pltpu.sync_copy(idx_hbm.at[pl.ds(base, ROWS_PER_W)], idx_v.at[pl.ds(0, ROWS_PER_W)])
    pltpu.sync_copy(ctab_hbm, ctab_v)

    def chunk_body(k, carry):
        row0 = base + k * CHUNK
        pltpu.sync_copy(pos_hbm.at[pl.ds(row0, CHUNK)], pos_v)

        def row_body(r, c2):
            gr = k * CHUNK + r
            ridx = idx_v[pl.ds(gr, NLANES)][0]
            for j in range(D_MODEL // NLANES):
                cond = ctab_v[ridx, pl.ds(j * NLANES, NLANES)]
                pos_v[r, pl.ds(j * NLANES, NLANES)] += cond
            return c2

        lax.fori_loop(0, CHUNK, row_body, 0)
        pltpu.sync_copy(pos_v, comb_hbm.at[pl.ds(row0, CHUNK)])
        return carry

    lax.fori_loop(0, NCHUNK, chunk_body, 0)


BL = 512
NB = MAX_LEN // BL


def _dense_body(in_ref, comb_ref, out_ref):
    out_ref[...] = in_ref[...] + comb_ref[...][None, :, :]


@jax.jit
def _dense_add(inputs, combined):
    return pl.pallas_call(
        _dense_body,
        grid=(NB,),
        in_specs=[
            pl.BlockSpec((BATCH, BL, D_MODEL), lambda i: (0, i, 0)),
            pl.BlockSpec((BL, D_MODEL), lambda i: (i, 0)),
        ],
        out_specs=pl.BlockSpec((BATCH, BL, D_MODEL), lambda i: (0, i, 0)),
        out_shape=jax.ShapeDtypeStruct((BATCH, MAX_LEN, D_MODEL), jnp.float32),
    )(inputs, combined)


def kernel(inputs, cond_pos, pos_table, cond_table):
    combined = _sc_combine(cond_pos, pos_table, cond_table)
    return _dense_add(inputs, combined)
